# baseline (speedup 1.0000x reference)
import functools

import jax
import jax.numpy as jnp
from jax import lax
from jax.experimental import pallas as pl
from jax.experimental.pallas import tpu as pltpu
from jax.experimental.pallas import tpu_sc as plsc

NC = 2
NS = 16
L = 16

_GDN = lax.GatherDimensionNumbers(
    offset_dims=(), collapsed_slice_dims=(0,), start_index_map=(0,))


def _bcast_lane(v, k):
    idx = jnp.full((L, 1), k, jnp.int32)
    return lax.gather(v, idx, _GDN, (1,),
                      mode=lax.GatherScatterMode.PROMISE_IN_BOUNDS)


def _sc_edge_pass(N, E, ZW, NZB, C):
    n_chunks = E // C
    assert n_chunks * C == E
    W = NC * NS
    rows_pt = N // NS
    assert rows_pt * NS == N
    nzones = ZW // L
    mesh = plsc.VectorSubcoreMesh(core_axis_name="c", subcore_axis_name="s")

    @functools.partial(
        pl.kernel,
        out_type=jax.ShapeDtypeStruct((NC, N, ZW), jnp.float32),
        mesh=mesh,
        compiler_params=pltpu.CompilerParams(use_tc_tiling_on_sc=False),
        scratch_types=[
            pltpu.VMEM((3, 2, C), jnp.int32),
            pltpu.VMEM((2, C, ZW), jnp.float32),
            pltpu.VMEM((2, C, L), jnp.float32),
            pltpu.VMEM_SHARED((N, ZW), jnp.float32),
            pltpu.SemaphoreType.DMA,
            pltpu.SemaphoreType.DMA,
            pltpu.SemaphoreType.DMA,
            pltpu.SemaphoreType.DMA,
            pltpu.SemaphoreType.DMA,
            pltpu.SemaphoreType.DMA,
        ],
    )
    def kern(ei, za, sbd, acc_out,
             idx_r, za_r, sbd_r, acc_sh,
             si0, si1, si2, sg0, sg1, ss):
        sem_i = (si0, si1, si2)
        sem_g = (sg0, sg1)
        cid = lax.axis_index("c")
        sid = lax.axis_index("s")
        wid = cid * NS + sid
        zero = jnp.zeros((L,), jnp.float32)

        @plsc.parallel_loop(0, C, 1, unroll=2)
        def _(r):
            for kk in range(nzones):
                za_r[0, r, pl.ds(kk * L, L)] = zero

        row0 = sid * rows_pt
        full, rem = divmod(rows_pt, C)
        for j in range(full):
            pltpu.sync_copy(za_r.at[0], acc_sh.at[pl.ds(row0 + j * C, C), :])
        if rem:
            pltpu.sync_copy(za_r.at[0, pl.ds(0, rem), :],
                            acc_sh.at[pl.ds(row0 + full * C, rem), :])
        plsc.subcore_barrier()

        nch = (n_chunks - wid + W - 1) // W

        def issue_idx(j, s3):
            off = pl.multiple_of((j * W + wid) * C, C)
            pltpu.async_copy(ei.at[:, pl.ds(off, C)], idx_r.at[s3], sem_i[s3])

        def wait_idx(s3):
            pltpu.make_async_copy(ei.at[:, pl.ds(0, C)], idx_r.at[s3],
                                  sem_i[s3]).wait()

        def issue_gather(s2, s3):
            pltpu.async_copy(za.at[idx_r.at[s3, 0]], za_r.at[s2], sem_g[s2])
            pltpu.async_copy(sbd.at[idx_r.at[s3, 1]], sbd_r.at[s2], sem_g[s2])

        def wait_gather(s2, s3):
            pltpu.make_async_copy(za.at[idx_r.at[s3, 0]], za_r.at[s2],
                                  sem_g[s2]).wait()
            pltpu.make_async_copy(sbd.at[idx_r.at[s3, 1]], sbd_r.at[s2],
                                  sem_g[s2]).wait()

        def issue_scatter(s2, s3):
            pltpu.async_copy(za_r.at[s2], acc_sh.at[idx_r.at[s3, 1]], ss,
                             add=True)

        def wait_scatter():
            pltpu.make_async_copy(za_r.at[0], acc_sh.at[idx_r.at[0, 1]],
                                  ss).wait()

        def compute(s2):
            @plsc.parallel_loop(0, C, 1, unroll=4)
            def _(e):
                x = za_r[s2, e, pl.ds((NZB + 1) * L, L)] + sbd_r[s2, e]
                ex = jnp.exp(jnp.maximum(x, 0.2 * x))
                for k in range(NZB):
                    g = _bcast_lane(ex, k)
                    za_r[s2, e, pl.ds(k * L, L)] = (
                        za_r[s2, e, pl.ds(k * L, L)] * g)
                za_r[s2, e, pl.ds(NZB * L, L)] = ex

        issue_idx(0, 0)
        issue_idx(1, 1)
        wait_idx(0)
        issue_gather(0, 0)

        def wave(w, carry):
            jbase = w * 6
            for s in range(6):
                j = jbase + s
                s2, s3 = s % 2, s % 3
                s2n, s3n = (s + 1) % 2, (s + 1) % 3
                s3nn = (s + 2) % 3

                @pl.when(j < nch)
                def _():
                    @pl.when(j >= 1)
                    def _():
                        wait_scatter()

                    @pl.when(j + 1 < nch)
                    def _():
                        wait_idx(s3n)
                        issue_gather(s2n, s3n)

                    wait_gather(s2, s3)
                    compute(s2)
                    issue_scatter(s2, s3)

                    @pl.when(j + 2 < nch)
                    def _():
                        issue_idx(j + 2, s3nn)
            return carry

        lax.fori_loop(0, (nch + 5) // 6, wave, 0)
        wait_scatter()
        plsc.subcore_barrier()
        pltpu.sync_copy(acc_sh.at[pl.ds(row0, rows_pt), :],
                        acc_out.at[cid, pl.ds(row0, rows_pt), :])

    return kern


def _tc_proj1(N, R):
    def kern(h_ref, w_ref, as_ref, ad_ref, za_ref, sd_ref):
        z = jnp.dot(h_ref[...], w_ref[...], preferred_element_type=jnp.float32)
        ss = jnp.dot(z, as_ref[...], preferred_element_type=jnp.float32)
        sd_ref[...] = jnp.dot(z, ad_ref[...], preferred_element_type=jnp.float32)
        za_ref[...] = jnp.concatenate(
            [z, jnp.ones((R, 8), jnp.float32), jnp.zeros((R, 8), jnp.float32),
             ss], axis=1)

    return pl.pallas_call(
        kern,
        grid=(N // R,),
        in_specs=[
            pl.BlockSpec((R, 128), lambda i: (i, 0)),
            pl.BlockSpec((128, 128), lambda i: (0, 0)),
            pl.BlockSpec((128, 16), lambda i: (0, 0)),
            pl.BlockSpec((128, 16), lambda i: (0, 0)),
        ],
        out_specs=[
            pl.BlockSpec((R, 160), lambda i: (i, 0)),
            pl.BlockSpec((R, 16), lambda i: (i, 0)),
        ],
        out_shape=[
            jax.ShapeDtypeStruct((N, 160), jnp.float32),
            jax.ShapeDtypeStruct((N, 16), jnp.float32),
        ],
    )


def _tc_mid(N, R):
    def kern(acc_ref, w2_ref, a2s_ref, a2d_ref, rep_ref, za2_ref, sd2_ref):
        asum = acc_ref[0] + acc_ref[1]
        den = asum[:, 128:136]
        denx = jnp.dot(den, rep_ref[...],
                       preferred_element_type=jnp.float32)
        h1 = asum[:, :128] / (denx + 1e-9)
        h1 = jnp.where(h1 > 0, h1, jnp.exp(h1) - 1.0)
        z2 = jnp.dot(h1, w2_ref[...], preferred_element_type=jnp.float32)
        ss2 = jnp.dot(z2, a2s_ref[...], preferred_element_type=jnp.float32)
        sd2_ref[...] = jnp.dot(z2, a2d_ref[...],
                               preferred_element_type=jnp.float32)
        za2_ref[...] = jnp.concatenate(
            [z2, jnp.ones((R, 1), jnp.float32),
             jnp.zeros((R, 15), jnp.float32), ss2], axis=1)

    return pl.pallas_call(
        kern,
        grid=(N // R,),
        in_specs=[
            pl.BlockSpec((2, R, 160), lambda i: (0, i, 0)),
            pl.BlockSpec((128, 16), lambda i: (0, 0)),
            pl.BlockSpec((16, 16), lambda i: (0, 0)),
            pl.BlockSpec((16, 16), lambda i: (0, 0)),
            pl.BlockSpec((8, 128), lambda i: (0, 0)),
        ],
        out_specs=[
            pl.BlockSpec((R, 48), lambda i: (i, 0)),
            pl.BlockSpec((R, 16), lambda i: (i, 0)),
        ],
        out_shape=[
            jax.ShapeDtypeStruct((N, 48), jnp.float32),
            jax.ShapeDtypeStruct((N, 16), jnp.float32),
        ],
    )


def _tc_final(N, R):
    def kern(acc_ref, out_ref):
        asum = acc_ref[0] + acc_ref[1]
        out_ref[...] = asum[:, :16] / (asum[:, 16:17] + 1e-9)

    return pl.pallas_call(
        kern,
        grid=(N // R,),
        in_specs=[pl.BlockSpec((2, R, 48), lambda i: (0, i, 0))],
        out_specs=pl.BlockSpec((R, 16), lambda i: (i, 0)),
        out_shape=jax.ShapeDtypeStruct((N, 16), jnp.float32),
    )


def kernel(h, edge_index, W1, a1, W2, a2):
    N, IN = h.shape
    E = edge_index.shape[1]
    HEADS, _, HID = W1.shape
    OUT = W2.shape[1]

    W1c = W1.transpose(1, 0, 2).reshape(IN, HEADS * HID)
    eye = jnp.eye(HEADS, dtype=jnp.float32)
    A1s = (a1[:, :HID][:, :, None] * eye[:, None, :]).reshape(HEADS * HID, HEADS)
    A1s = jnp.concatenate([A1s, jnp.zeros((HEADS * HID, L - HEADS), jnp.float32)], 1)
    A1d = (a1[:, HID:][:, :, None] * eye[:, None, :]).reshape(HEADS * HID, HEADS)
    A1d = jnp.concatenate([A1d, jnp.zeros((HEADS * HID, L - HEADS), jnp.float32)], 1)
    A2s = jnp.concatenate([a2[:OUT][:, None], jnp.zeros((OUT, L - 1), jnp.float32)], 1)
    A2d = jnp.concatenate([a2[OUT:][:, None], jnp.zeros((OUT, L - 1), jnp.float32)], 1)
    REP = jnp.kron(eye, jnp.ones((1, HID), jnp.float32))

    R = 1000
    za1, sbd1 = _tc_proj1(N, R)(h, W1c, A1s, A1d)
    ei = edge_index.astype(jnp.int32)
    acc1 = _sc_edge_pass(N, E, HEADS * HID + 2 * L, HEADS, 80)(ei, za1, sbd1)
    za2, sbd2 = _tc_mid(N, R)(acc1, W2, A2s, A2d, REP)
    acc2 = _sc_edge_pass(N, E, OUT + 2 * L, 1, 128)(ei, za2, sbd2)
    return _tc_final(N, R)(acc2)


# speedup vs baseline: 1.0504x; 1.0504x over previous
import functools

import jax
import jax.numpy as jnp
from jax import lax
from jax.experimental import pallas as pl
from jax.experimental.pallas import tpu as pltpu
from jax.experimental.pallas import tpu_sc as plsc

NC = 2
NS = 16
L = 16

_GDN = lax.GatherDimensionNumbers(
    offset_dims=(), collapsed_slice_dims=(0,), start_index_map=(0,))


def _bcast_lane(v, k):
    idx = jnp.full((L, 1), k, jnp.int32)
    return lax.gather(v, idx, _GDN, (1,),
                      mode=lax.GatherScatterMode.PROMISE_IN_BOUNDS)


def _sc_edge_pass(N, E, ZW, NZB, C):
    n_chunks = E // C
    assert n_chunks * C == E
    W = NC * NS
    rows_pt = N // NS
    assert rows_pt * NS == N
    nzones = ZW // L
    mesh = plsc.VectorSubcoreMesh(core_axis_name="c", subcore_axis_name="s")

    @functools.partial(
        pl.kernel,
        out_type=jax.ShapeDtypeStruct((NC, N, ZW), jnp.float32),
        mesh=mesh,
        compiler_params=pltpu.CompilerParams(use_tc_tiling_on_sc=False),
        scratch_types=[
            pltpu.VMEM((3, 2, C), jnp.int32),
            pltpu.VMEM((2, C, ZW), jnp.float32),
            pltpu.VMEM((2, C, L), jnp.float32),
            pltpu.VMEM_SHARED((N, ZW), jnp.float32),
            pltpu.SemaphoreType.DMA,
            pltpu.SemaphoreType.DMA,
            pltpu.SemaphoreType.DMA,
            pltpu.SemaphoreType.DMA,
            pltpu.SemaphoreType.DMA,
            pltpu.SemaphoreType.DMA,
        ],
    )
    def kern(ei, za, sbd, acc_out,
             idx_r, za_r, sbd_r, acc_sh,
             si0, si1, si2, sg0, sg1, ss):
        sem_i = (si0, si1, si2)
        sem_g = (sg0, sg1)
        cid = lax.axis_index("c")
        sid = lax.axis_index("s")
        wid = cid * NS + sid
        zero = jnp.zeros((L,), jnp.float32)

        @plsc.parallel_loop(0, C, 1, unroll=2)
        def _(r):
            for kk in range(nzones):
                za_r[0, r, pl.ds(kk * L, L)] = zero

        row0 = sid * rows_pt
        full, rem = divmod(rows_pt, C)
        for j in range(full):
            pltpu.sync_copy(za_r.at[0], acc_sh.at[pl.ds(row0 + j * C, C), :])
        if rem:
            pltpu.sync_copy(za_r.at[0, pl.ds(0, rem), :],
                            acc_sh.at[pl.ds(row0 + full * C, rem), :])
        plsc.subcore_barrier()

        nch = (n_chunks - wid + W - 1) // W

        def issue_idx(j, s3):
            off = pl.multiple_of((j * W + wid) * C, C)
            pltpu.async_copy(ei.at[:, pl.ds(off, C)], idx_r.at[s3], sem_i[s3])

        def wait_idx(s3):
            pltpu.make_async_copy(ei.at[:, pl.ds(0, C)], idx_r.at[s3],
                                  sem_i[s3]).wait()

        def issue_gather(s2, s3):
            pltpu.async_copy(za.at[idx_r.at[s3, 0]], za_r.at[s2], sem_g[s2])
            pltpu.async_copy(sbd.at[idx_r.at[s3, 1]], sbd_r.at[s2], sem_g[s2])

        def wait_gather(s2, s3):
            pltpu.make_async_copy(za.at[idx_r.at[s3, 0]], za_r.at[s2],
                                  sem_g[s2]).wait()
            pltpu.make_async_copy(sbd.at[idx_r.at[s3, 1]], sbd_r.at[s2],
                                  sem_g[s2]).wait()

        def issue_scatter(s2, s3):
            pltpu.async_copy(za_r.at[s2], acc_sh.at[idx_r.at[s3, 1]], ss,
                             add=True)

        def wait_scatter():
            pltpu.make_async_copy(za_r.at[0], acc_sh.at[idx_r.at[0, 1]],
                                  ss).wait()

        def compute(s2):
            @plsc.parallel_loop(0, C, 1, unroll=4)
            def _(e):
                x = za_r[s2, e, pl.ds(NZB * L, L)] + sbd_r[s2, e]
                ex = jnp.exp(jnp.maximum(x, 0.2 * x))
                for k in range(NZB):
                    g = _bcast_lane(ex, k)
                    za_r[s2, e, pl.ds(k * L, L)] = (
                        za_r[s2, e, pl.ds(k * L, L)] * g)
                za_r[s2, e, pl.ds(NZB * L, L)] = ex

        issue_idx(0, 0)
        issue_idx(1, 1)
        wait_idx(0)
        issue_gather(0, 0)

        def wave(w, carry):
            jbase = w * 6
            for s in range(6):
                j = jbase + s
                s2, s3 = s % 2, s % 3
                s2n, s3n = (s + 1) % 2, (s + 1) % 3
                s3nn = (s + 2) % 3

                @pl.when(j < nch)
                def _():
                    @pl.when(j >= 1)
                    def _():
                        wait_scatter()

                    @pl.when(j + 1 < nch)
                    def _():
                        wait_idx(s3n)
                        issue_gather(s2n, s3n)

                    wait_gather(s2, s3)
                    compute(s2)
                    issue_scatter(s2, s3)

                    @pl.when(j + 2 < nch)
                    def _():
                        issue_idx(j + 2, s3nn)
            return carry

        lax.fori_loop(0, (nch + 5) // 6, wave, 0)
        wait_scatter()
        plsc.subcore_barrier()
        pltpu.sync_copy(acc_sh.at[pl.ds(row0, rows_pt), :],
                        acc_out.at[cid, pl.ds(row0, rows_pt), :])

    return kern


def _tc_proj1(N, R):
    def kern(h_ref, w_ref, as_ref, ad_ref, za_ref, sd_ref):
        z = jnp.dot(h_ref[...], w_ref[...], preferred_element_type=jnp.float32)
        ss = jnp.dot(z, as_ref[...], preferred_element_type=jnp.float32)
        sd_ref[...] = jnp.dot(z, ad_ref[...], preferred_element_type=jnp.float32)
        za_ref[...] = jnp.concatenate([z, ss], axis=1)

    return pl.pallas_call(
        kern,
        grid=(N // R,),
        in_specs=[
            pl.BlockSpec((R, 128), lambda i: (i, 0)),
            pl.BlockSpec((128, 128), lambda i: (0, 0)),
            pl.BlockSpec((128, 16), lambda i: (0, 0)),
            pl.BlockSpec((128, 16), lambda i: (0, 0)),
        ],
        out_specs=[
            pl.BlockSpec((R, 144), lambda i: (i, 0)),
            pl.BlockSpec((R, 16), lambda i: (i, 0)),
        ],
        out_shape=[
            jax.ShapeDtypeStruct((N, 144), jnp.float32),
            jax.ShapeDtypeStruct((N, 16), jnp.float32),
        ],
    )


def _tc_mid(N, R):
    def kern(acc_ref, w2_ref, a2s_ref, a2d_ref, rep_ref, za2_ref, sd2_ref):
        asum = acc_ref[0] + acc_ref[1]
        den = asum[:, 128:136]
        denx = jnp.dot(den, rep_ref[...],
                       preferred_element_type=jnp.float32)
        h1 = asum[:, :128] / (denx + 1e-9)
        h1 = jnp.where(h1 > 0, h1, jnp.exp(h1) - 1.0)
        z2 = jnp.dot(h1, w2_ref[...], preferred_element_type=jnp.float32)
        ss2 = jnp.dot(z2, a2s_ref[...], preferred_element_type=jnp.float32)
        sd2_ref[...] = jnp.dot(z2, a2d_ref[...],
                               preferred_element_type=jnp.float32)
        za2_ref[...] = jnp.concatenate([z2, ss2], axis=1)

    return pl.pallas_call(
        kern,
        grid=(N // R,),
        in_specs=[
            pl.BlockSpec((2, R, 144), lambda i: (0, i, 0)),
            pl.BlockSpec((128, 16), lambda i: (0, 0)),
            pl.BlockSpec((16, 16), lambda i: (0, 0)),
            pl.BlockSpec((16, 16), lambda i: (0, 0)),
            pl.BlockSpec((8, 128), lambda i: (0, 0)),
        ],
        out_specs=[
            pl.BlockSpec((R, 32), lambda i: (i, 0)),
            pl.BlockSpec((R, 16), lambda i: (i, 0)),
        ],
        out_shape=[
            jax.ShapeDtypeStruct((N, 32), jnp.float32),
            jax.ShapeDtypeStruct((N, 16), jnp.float32),
        ],
    )


def _tc_final(N, R):
    def kern(acc_ref, out_ref):
        asum = acc_ref[0] + acc_ref[1]
        out_ref[...] = asum[:, :16] / (asum[:, 16:17] + 1e-9)

    return pl.pallas_call(
        kern,
        grid=(N // R,),
        in_specs=[pl.BlockSpec((2, R, 32), lambda i: (0, i, 0))],
        out_specs=pl.BlockSpec((R, 16), lambda i: (i, 0)),
        out_shape=jax.ShapeDtypeStruct((N, 16), jnp.float32),
    )


def kernel(h, edge_index, W1, a1, W2, a2):
    N, IN = h.shape
    E = edge_index.shape[1]
    HEADS, _, HID = W1.shape
    OUT = W2.shape[1]

    W1c = W1.transpose(1, 0, 2).reshape(IN, HEADS * HID)
    eye = jnp.eye(HEADS, dtype=jnp.float32)
    A1s = (a1[:, :HID][:, :, None] * eye[:, None, :]).reshape(HEADS * HID, HEADS)
    A1s = jnp.concatenate([A1s, jnp.zeros((HEADS * HID, L - HEADS), jnp.float32)], 1)
    A1d = (a1[:, HID:][:, :, None] * eye[:, None, :]).reshape(HEADS * HID, HEADS)
    A1d = jnp.concatenate([A1d, jnp.zeros((HEADS * HID, L - HEADS), jnp.float32)], 1)
    A2s = jnp.concatenate([a2[:OUT][:, None], jnp.zeros((OUT, L - 1), jnp.float32)], 1)
    A2d = jnp.concatenate([a2[OUT:][:, None], jnp.zeros((OUT, L - 1), jnp.float32)], 1)
    REP = jnp.kron(eye, jnp.ones((1, HID), jnp.float32))

    R = 1000
    za1, sbd1 = _tc_proj1(N, R)(h, W1c, A1s, A1d)
    ei = edge_index.astype(jnp.int32)
    acc1 = _sc_edge_pass(N, E, HEADS * HID + L, HEADS, 80)(ei, za1, sbd1)
    za2, sbd2 = _tc_mid(N, R)(acc1, W2, A2s, A2d, REP)
    acc2 = _sc_edge_pass(N, E, OUT + L, 1, 128)(ei, za2, sbd2)
    return _tc_final(N, R)(acc2)


# speedup vs baseline: 1.0915x; 1.0391x over previous
import functools

import jax
import jax.numpy as jnp
from jax import lax
from jax.experimental import pallas as pl
from jax.experimental.pallas import tpu as pltpu
from jax.experimental.pallas import tpu_sc as plsc

NC = 2
NS = 16
L = 16

_GDN = lax.GatherDimensionNumbers(
    offset_dims=(), collapsed_slice_dims=(0,), start_index_map=(0,))


def _bcast_lane(v, k):
    idx = jnp.full((L, 1), k, jnp.int32)
    return lax.gather(v, idx, _GDN, (1,),
                      mode=lax.GatherScatterMode.PROMISE_IN_BOUNDS)


def _sc_edge_pass(N, E, ZW, NZB, C, packed=False):
    n_chunks = E // C
    assert n_chunks * C == E
    W = NC * NS
    rows_pt = N // NS
    assert rows_pt * NS == N
    nzones = ZW // L
    PW = NZB * 8 + 16
    mesh = plsc.VectorSubcoreMesh(core_axis_name="c", subcore_axis_name="s")

    @functools.partial(
        pl.kernel,
        out_type=jax.ShapeDtypeStruct((NC, N, ZW), jnp.float32),
        mesh=mesh,
        compiler_params=pltpu.CompilerParams(use_tc_tiling_on_sc=False),
        scratch_types=[
            pltpu.VMEM((3, 2, C), jnp.int32),
            pltpu.VMEM((2, C, PW) if packed else (2, C, 8),
                       jnp.int32),
            pltpu.VMEM((2, C, ZW), jnp.float32),
            pltpu.VMEM((2, C, L), jnp.float32),
            pltpu.VMEM_SHARED((N, ZW), jnp.float32),
            pltpu.SemaphoreType.DMA,
            pltpu.SemaphoreType.DMA,
            pltpu.SemaphoreType.DMA,
            pltpu.SemaphoreType.DMA,
            pltpu.SemaphoreType.DMA,
            pltpu.SemaphoreType.DMA,
        ],
    )
    def kern(ei, za, sbd, acc_out,
             idx_r, za_g, za_r, sbd_r, acc_sh,
             si0, si1, si2, sg0, sg1, ss):
        sem_i = (si0, si1, si2)
        sem_g = (sg0, sg1)
        cid = lax.axis_index("c")
        sid = lax.axis_index("s")
        wid = cid * NS + sid
        zero = jnp.zeros((L,), jnp.float32)

        @plsc.parallel_loop(0, C, 1, unroll=2)
        def _(r):
            for kk in range(nzones):
                za_r[0, r, pl.ds(kk * L, L)] = zero

        row0 = sid * rows_pt
        full, rem = divmod(rows_pt, C)
        for j in range(full):
            pltpu.sync_copy(za_r.at[0], acc_sh.at[pl.ds(row0 + j * C, C), :])
        if rem:
            pltpu.sync_copy(za_r.at[0, pl.ds(0, rem), :],
                            acc_sh.at[pl.ds(row0 + full * C, rem), :])
        plsc.subcore_barrier()

        nch = (n_chunks - wid + W - 1) // W

        def issue_idx(j, s3):
            off = pl.multiple_of((j * W + wid) * C, C)
            pltpu.async_copy(ei.at[:, pl.ds(off, C)], idx_r.at[s3], sem_i[s3])

        def wait_idx(s3):
            pltpu.make_async_copy(ei.at[:, pl.ds(0, C)], idx_r.at[s3],
                                  sem_i[s3]).wait()

        gdst = za_g if packed else za_r

        def issue_gather(s2, s3):
            pltpu.async_copy(za.at[idx_r.at[s3, 0]], gdst.at[s2], sem_g[s2])
            pltpu.async_copy(sbd.at[idx_r.at[s3, 1]], sbd_r.at[s2], sem_g[s2])

        def wait_gather(s2, s3):
            pltpu.make_async_copy(za.at[idx_r.at[s3, 0]], gdst.at[s2],
                                  sem_g[s2]).wait()
            pltpu.make_async_copy(sbd.at[idx_r.at[s3, 1]], sbd_r.at[s2],
                                  sem_g[s2]).wait()

        def issue_scatter(s2, s3):
            pltpu.async_copy(za_r.at[s2], acc_sh.at[idx_r.at[s3, 1]], ss,
                             add=True)

        def wait_scatter():
            pltpu.make_async_copy(za_r.at[0], acc_sh.at[idx_r.at[0, 1]],
                                  ss).wait()

        def compute(s2):
            @plsc.parallel_loop(0, C, 1, unroll=4)
            def _(e):
                if packed:
                    sc_words = za_g[s2, e, pl.ds(NZB * 8, L)]
                    x = (lax.bitcast_convert_type(sc_words, jnp.float32)
                         + sbd_r[s2, e])
                else:
                    x = za_r[s2, e, pl.ds(NZB * L, L)] + sbd_r[s2, e]
                ex = jnp.exp(jnp.maximum(x, 0.2 * x))
                if packed:
                    for k in range(NZB // 2):
                        w = za_g[s2, e, pl.ds(16 * k, 16)]
                        zlo = lax.bitcast_convert_type(
                            lax.shift_left(w, 16), jnp.float32)
                        zhi = lax.bitcast_convert_type(
                            lax.bitwise_and(w, jnp.int32(-65536)),
                            jnp.float32)
                        za_r[s2, e, pl.ds(32 * k, L)] = (
                            zlo * _bcast_lane(ex, 2 * k))
                        za_r[s2, e, pl.ds(32 * k + L, L)] = (
                            zhi * _bcast_lane(ex, 2 * k + 1))
                else:
                    for k in range(NZB):
                        g = _bcast_lane(ex, k)
                        za_r[s2, e, pl.ds(k * L, L)] = (
                            za_r[s2, e, pl.ds(k * L, L)] * g)
                za_r[s2, e, pl.ds(NZB * L, L)] = ex

        issue_idx(0, 0)
        issue_idx(1, 1)
        wait_idx(0)
        issue_gather(0, 0)

        def wave(w, carry):
            jbase = w * 6
            for s in range(6):
                j = jbase + s
                s2, s3 = s % 2, s % 3
                s2n, s3n = (s + 1) % 2, (s + 1) % 3
                s3nn = (s + 2) % 3

                @pl.when(j < nch)
                def _():
                    @pl.when(j >= 1)
                    def _():
                        wait_scatter()

                    @pl.when(j + 1 < nch)
                    def _():
                        wait_idx(s3n)
                        issue_gather(s2n, s3n)

                    wait_gather(s2, s3)
                    compute(s2)
                    issue_scatter(s2, s3)

                    @pl.when(j + 2 < nch)
                    def _():
                        issue_idx(j + 2, s3nn)
            return carry

        lax.fori_loop(0, (nch + 5) // 6, wave, 0)
        wait_scatter()
        plsc.subcore_barrier()
        pltpu.sync_copy(acc_sh.at[pl.ds(row0, rows_pt), :],
                        acc_out.at[cid, pl.ds(row0, rows_pt), :])

    return kern


def _tc_proj1(N, R):
    def kern(h_ref, w_ref, as_ref, ad_ref, lo_ref, hi_ref, za_ref, sd_ref):
        z = jnp.dot(h_ref[...], w_ref[...], preferred_element_type=jnp.float32)
        ss = jnp.dot(z, as_ref[...], preferred_element_type=jnp.float32)
        sd_ref[...] = jnp.dot(z, ad_ref[...], preferred_element_type=jnp.float32)
        zlo = jnp.dot(z, lo_ref[...], preferred_element_type=jnp.float32)
        zhi = jnp.dot(z, hi_ref[...], preferred_element_type=jnp.float32)
        lo16 = lax.bitcast_convert_type(
            zlo.astype(jnp.bfloat16), jnp.uint16).astype(jnp.uint32)
        hi16 = lax.bitcast_convert_type(
            zhi.astype(jnp.bfloat16), jnp.uint16).astype(jnp.uint32)
        packed = lax.bitcast_convert_type(
            jnp.bitwise_or(jnp.left_shift(hi16, 16), lo16), jnp.int32)
        za_ref[...] = jnp.concatenate(
            [packed, lax.bitcast_convert_type(ss, jnp.int32)], axis=1)

    return pl.pallas_call(
        kern,
        grid=(N // R,),
        in_specs=[
            pl.BlockSpec((R, 128), lambda i: (i, 0)),
            pl.BlockSpec((128, 128), lambda i: (0, 0)),
            pl.BlockSpec((128, 16), lambda i: (0, 0)),
            pl.BlockSpec((128, 16), lambda i: (0, 0)),
            pl.BlockSpec((128, 64), lambda i: (0, 0)),
            pl.BlockSpec((128, 64), lambda i: (0, 0)),
        ],
        out_specs=[
            pl.BlockSpec((R, 80), lambda i: (i, 0)),
            pl.BlockSpec((R, 16), lambda i: (i, 0)),
        ],
        out_shape=[
            jax.ShapeDtypeStruct((N, 80), jnp.int32),
            jax.ShapeDtypeStruct((N, 16), jnp.float32),
        ],
    )


def _tc_mid(N, R):
    def kern(acc_ref, w2_ref, a2s_ref, a2d_ref, rep_ref, za2_ref, sd2_ref):
        asum = acc_ref[0] + acc_ref[1]
        den = asum[:, 128:136]
        denx = jnp.dot(den, rep_ref[...],
                       preferred_element_type=jnp.float32)
        h1 = asum[:, :128] / (denx + 1e-9)
        h1 = jnp.where(h1 > 0, h1, jnp.exp(h1) - 1.0)
        z2 = jnp.dot(h1, w2_ref[...], preferred_element_type=jnp.float32)
        ss2 = jnp.dot(z2, a2s_ref[...], preferred_element_type=jnp.float32)
        sd2_ref[...] = jnp.dot(z2, a2d_ref[...],
                               preferred_element_type=jnp.float32)
        za2_ref[...] = jnp.concatenate([z2, ss2], axis=1)

    return pl.pallas_call(
        kern,
        grid=(N // R,),
        in_specs=[
            pl.BlockSpec((2, R, 144), lambda i: (0, i, 0)),
            pl.BlockSpec((128, 16), lambda i: (0, 0)),
            pl.BlockSpec((16, 16), lambda i: (0, 0)),
            pl.BlockSpec((16, 16), lambda i: (0, 0)),
            pl.BlockSpec((8, 128), lambda i: (0, 0)),
        ],
        out_specs=[
            pl.BlockSpec((R, 32), lambda i: (i, 0)),
            pl.BlockSpec((R, 16), lambda i: (i, 0)),
        ],
        out_shape=[
            jax.ShapeDtypeStruct((N, 32), jnp.float32),
            jax.ShapeDtypeStruct((N, 16), jnp.float32),
        ],
    )


def _tc_final(N, R):
    def kern(acc_ref, out_ref):
        asum = acc_ref[0] + acc_ref[1]
        out_ref[...] = asum[:, :16] / (asum[:, 16:17] + 1e-9)

    return pl.pallas_call(
        kern,
        grid=(N // R,),
        in_specs=[pl.BlockSpec((2, R, 32), lambda i: (0, i, 0))],
        out_specs=pl.BlockSpec((R, 16), lambda i: (i, 0)),
        out_shape=jax.ShapeDtypeStruct((N, 16), jnp.float32),
    )


def kernel(h, edge_index, W1, a1, W2, a2):
    N, IN = h.shape
    E = edge_index.shape[1]
    HEADS, _, HID = W1.shape
    OUT = W2.shape[1]

    W1c = W1.transpose(1, 0, 2).reshape(IN, HEADS * HID)
    eye = jnp.eye(HEADS, dtype=jnp.float32)
    A1s = (a1[:, :HID][:, :, None] * eye[:, None, :]).reshape(HEADS * HID, HEADS)
    A1s = jnp.concatenate([A1s, jnp.zeros((HEADS * HID, L - HEADS), jnp.float32)], 1)
    A1d = (a1[:, HID:][:, :, None] * eye[:, None, :]).reshape(HEADS * HID, HEADS)
    A1d = jnp.concatenate([A1d, jnp.zeros((HEADS * HID, L - HEADS), jnp.float32)], 1)
    A2s = jnp.concatenate([a2[:OUT][:, None], jnp.zeros((OUT, L - 1), jnp.float32)], 1)
    A2d = jnp.concatenate([a2[OUT:][:, None], jnp.zeros((OUT, L - 1), jnp.float32)], 1)
    REP = jnp.kron(eye, jnp.ones((1, HID), jnp.float32))
    cols = jnp.arange(64)
    rlo = 32 * (cols // 16) + (cols % 16)
    SelLo = (jnp.arange(128)[:, None] == rlo[None, :]).astype(jnp.float32)
    SelHi = (jnp.arange(128)[:, None] == (rlo + 16)[None, :]).astype(jnp.float32)

    R = 1000
    za1, sbd1 = _tc_proj1(N, R)(h, W1c, A1s, A1d, SelLo, SelHi)
    ei = edge_index.astype(jnp.int32)
    acc1 = _sc_edge_pass(N, E, HEADS * HID + L, HEADS, 80, packed=True)(
        ei, za1, sbd1)
    za2, sbd2 = _tc_mid(N, R)(acc1, W2, A2s, A2d, REP)
    acc2 = _sc_edge_pass(N, E, OUT + L, 1, 128)(ei, za2, sbd2)
    return _tc_final(N, R)(acc2)
